# baseline (speedup 1.0000x reference)
import math

import jax
import jax.numpy as jnp
from jax import lax
from jax.experimental import pallas as pl
from jax.experimental.pallas import tpu as pltpu
from jax.experimental.pallas import tpu_sc as plsc

D_MODEL = 64
D_PAD = 128
SCALE = math.sqrt(D_MODEL)
CHUNK = 128
BV = 128

_info = plsc.get_sparse_core_info()
NUM_CORES = _info.num_cores
NUM_SUBCORES = _info.num_subcores
NW = NUM_CORES * NUM_SUBCORES


def _make_sc_format(vocab: int):
    n_full = vocab // BV
    tail_v0 = n_full * BV
    tail_n = vocab - tail_v0
    n_blocks = 2 * ((n_full + 2 * NW - 1) // (2 * NW))
    last_v0 = (n_full - 1) * BV

    mesh = plsc.VectorSubcoreMesh(core_axis_name="c", subcore_axis_name="s")

    @pl.kernel(
        out_type=jax.ShapeDtypeStruct((vocab * D_PAD,), jnp.float32),
        mesh=mesh,
        compiler_params=pltpu.CompilerParams(needs_layout_passes=False),
        scratch_types=[
            pltpu.VMEM((D_MODEL, BV), jnp.float32),
            pltpu.VMEM((D_MODEL, BV), jnp.float32),
            pltpu.VMEM((BV * D_PAD,), jnp.float32),
            pltpu.VMEM((BV * D_PAD,), jnp.float32),
            pltpu.VMEM((tail_n, D_MODEL), jnp.float32),
            pltpu.SemaphoreType.DMA,
            pltpu.SemaphoreType.DMA,
            pltpu.SemaphoreType.DMA,
            pltpu.SemaphoreType.DMA,
        ],
    )
    def fmt(tt_hbm, tail_hbm, out_hbm,
            i0, i1, o0, o1, tbuf, lsem0, lsem1, ssem0, ssem1):
        wid = lax.axis_index("s") * NUM_CORES + lax.axis_index("c")

        ibuf = (i0, i1)
        obuf = (o0, o1)
        lsem = (lsem0, lsem1)
        ssem = (ssem0, ssem1)
        lane = lax.iota(jnp.int32, 16)

        def bid_of(j):
            return jnp.minimum(wid * n_blocks + j, n_full - 1)

        def load_start(j, b):
            pltpu.make_async_copy(
                tt_hbm.at[:, pl.ds(bid_of(j) * BV, BV)], ibuf[b], lsem[b]).start()

        def store(j, b):
            return pltpu.make_async_copy(
                obuf[b],
                out_hbm.at[pl.ds(bid_of(j) * (BV * D_PAD), BV * D_PAD)],
                ssem[b])

        flat_g = [(lane + g * 16) * D_PAD for g in range(BV // 16)]

        def transpose(b):
            src, dst = ibuf[b], obuf[b]
            for d in range(D_MODEL):
                for g in range(BV // 16):
                    v = src[d, pl.ds(g * 16, 16)]
                    plsc.store_scatter(dst, [flat_g[g] + d], v * SCALE)

        load_start(0, 0)
        load_start(1, 1)

        def pair(i, carry):
            for b in range(2):
                j = i * 2 + b
                pltpu.make_async_copy(
                    tt_hbm.at[:, pl.ds(bid_of(j) * BV, BV)], ibuf[b], lsem[b]).wait()

                @pl.when(i >= 1)
                def _wait_prev_store():
                    store(j - 2, b).wait()

                transpose(b)

                @pl.when(i < n_blocks // 2 - 1)
                def _next_load():
                    load_start(j + 2, b)

                store(j, b).start()
            return carry

        lax.fori_loop(0, n_blocks // 2, pair, 0)

        for b in range(2):
            store(n_blocks - 2 + b, b).wait()

        @pl.when(wid == 0)
        def _tail():
            pltpu.sync_copy(tail_hbm, tbuf)
            for u in range(tail_n):
                for c in range(D_MODEL // 16):
                    o0[pl.ds(u * D_PAD + c * 16, 16)] = (
                        tbuf[u, pl.ds(c * 16, 16)] * SCALE)
            pltpu.sync_copy(o0.at[pl.ds(0, tail_n * D_PAD)],
                            out_hbm.at[pl.ds(tail_v0 * D_PAD, tail_n * D_PAD)])

    return fmt


def _make_sc_lookup(n_chunks_total: int):
    n_chunks = n_chunks_total // NW
    assert n_chunks * NW == n_chunks_total and n_chunks % 2 == 0
    n_pairs = n_chunks // 2

    mesh = plsc.VectorSubcoreMesh(core_axis_name="c", subcore_axis_name="s")

    @pl.kernel(
        out_type=jax.ShapeDtypeStruct((n_chunks_total * CHUNK, D_MODEL),
                                      jnp.float32),
        mesh=mesh,
        scratch_types=[
            pltpu.VMEM((n_chunks, CHUNK), jnp.int32),
            pltpu.VMEM((CHUNK, D_PAD), jnp.float32),
            pltpu.VMEM((CHUNK, D_PAD), jnp.float32),
            pltpu.VMEM((CHUNK, D_MODEL), jnp.float32),
            pltpu.VMEM((CHUNK, D_MODEL), jnp.float32),
            pltpu.SemaphoreType.DMA,
            pltpu.SemaphoreType.DMA,
            pltpu.SemaphoreType.DMA,
            pltpu.SemaphoreType.DMA,
        ],
    )
    def lookup(idx_hbm, table_hbm, out_hbm,
               idx_v, g0, g1, s0, s1, gsem0, gsem1, ssem0, ssem1):
        wid = lax.axis_index("s") * NUM_CORES + lax.axis_index("c")

        pltpu.sync_copy(idx_hbm.at[pl.ds(wid * n_chunks, n_chunks)], idx_v)

        gbuf = (g0, g1)
        sbuf = (s0, s1)
        gsem = (gsem0, gsem1)
        ssem = (ssem0, ssem1)

        def gather_start(j, b):
            pltpu.make_async_copy(
                table_hbm.at[idx_v.at[j]], gbuf[b], gsem[b]).start()

        def compact(b):
            src, dst = gbuf[b], sbuf[b]

            def body(r, carry):
                base = r * 8
                for k in range(8):
                    for c in range(D_MODEL // 16):
                        v = src[base + k, pl.ds(c * 16, 16)]
                        dst[base + k, pl.ds(c * 16, 16)] = v
                return carry

            lax.fori_loop(0, CHUNK // 8, body, 0)

        def scatter(j, b):
            return pltpu.make_async_copy(
                sbuf[b],
                out_hbm.at[pl.ds((wid * n_chunks + j) * CHUNK, CHUNK)],
                ssem[b])

        gather_start(0, 0)
        gather_start(1, 1)

        def pair(i, carry):
            for b in range(2):
                j = i * 2 + b
                pltpu.make_async_copy(
                    table_hbm.at[idx_v.at[j]], gbuf[b], gsem[b]).wait()

                @pl.when(i >= 1)
                def _wait_prev_scatter():
                    scatter(j - 2, b).wait()

                compact(b)

                @pl.when(i < n_pairs - 1)
                def _next_gather():
                    gather_start(j + 2, b)

                scatter(j, b).start()
            return carry

        lax.fori_loop(0, n_pairs, pair, 0)

        for b in range(2):
            scatter(n_chunks - 2 + b, b).wait()

    return lookup


def kernel(x, table):
    b, s = x.shape
    n = b * s
    vocab, d = table.shape
    assert d == D_MODEL and n % (NW * CHUNK * 2) == 0, (b, s, table.shape)
    idx = x.reshape(n // CHUNK, CHUNK).astype(jnp.int32)
    fmt = _make_sc_format(vocab)
    tail_v0 = (vocab // BV) * BV
    tablep = fmt(table.T, table[tail_v0:]).reshape(vocab, D_PAD)
    lookup = _make_sc_lookup(n // CHUNK)
    out = lookup(idx, tablep)
    return out.reshape(b, s, D_MODEL)


# speedup vs baseline: 1.0196x; 1.0196x over previous
import math

import jax
import jax.numpy as jnp
from jax import lax
from jax.experimental import pallas as pl
from jax.experimental.pallas import tpu as pltpu
from jax.experimental.pallas import tpu_sc as plsc

D_MODEL = 64
D_PAD = 128
SCALE = math.sqrt(D_MODEL)
CHUNK = 128
BV = 128

_info = plsc.get_sparse_core_info()
NUM_CORES = _info.num_cores
NUM_SUBCORES = _info.num_subcores
NW = NUM_CORES * NUM_SUBCORES


def _make_sc_format(vocab: int):
    n_full = vocab // BV
    tail_v0 = n_full * BV
    tail_n = vocab - tail_v0
    n_blocks = 2 * ((n_full + 2 * NW - 1) // (2 * NW))
    last_v0 = (n_full - 1) * BV

    mesh = plsc.VectorSubcoreMesh(core_axis_name="c", subcore_axis_name="s")

    @pl.kernel(
        out_type=jax.ShapeDtypeStruct((vocab, D_PAD), jnp.float32),
        mesh=mesh,
        compiler_params=pltpu.CompilerParams(needs_layout_passes=False),
        scratch_types=[
            pltpu.VMEM((D_MODEL, BV), jnp.float32),
            pltpu.VMEM((D_MODEL, BV), jnp.float32),
            pltpu.VMEM((BV, D_PAD + 1), jnp.float32),
            pltpu.VMEM((BV, D_PAD + 1), jnp.float32),
            pltpu.VMEM((tail_n, D_MODEL), jnp.float32),
            pltpu.SemaphoreType.DMA,
            pltpu.SemaphoreType.DMA,
            pltpu.SemaphoreType.DMA,
            pltpu.SemaphoreType.DMA,
        ],
    )
    def fmt(tt_hbm, tail_hbm, out_hbm,
            i0, i1, o0, o1, tbuf, lsem0, lsem1, ssem0, ssem1):
        wid = lax.axis_index("s") * NUM_CORES + lax.axis_index("c")

        ibuf = (i0, i1)
        obuf = (o0, o1)
        lsem = (lsem0, lsem1)
        ssem = (ssem0, ssem1)
        lane = lax.iota(jnp.int32, 16)

        def bid_of(j):
            return jnp.minimum(wid * n_blocks + j, n_full - 1)

        def load_start(j, b):
            pltpu.make_async_copy(
                tt_hbm.at[:, pl.ds(bid_of(j) * BV, BV)], ibuf[b], lsem[b]).start()

        def store(j, b):
            return pltpu.make_async_copy(
                obuf[b].at[:, pl.ds(0, D_PAD)],
                out_hbm.at[pl.ds(bid_of(j) * BV, BV)], ssem[b])

        row_g = [lane + g * 16 for g in range(BV // 16)]

        def transpose(b):
            src, dst = ibuf[b], obuf[b]

            def body(r, carry):
                for k in range(8):
                    d = r * 8 + k
                    col = lane * 0 + d
                    for g in range(BV // 16):
                        v = src[d, pl.ds(g * 16, 16)]
                        plsc.store_scatter(dst, [row_g[g], col], v * SCALE)
                return carry

            lax.fori_loop(0, D_MODEL // 8, body, 0)

        load_start(0, 0)
        load_start(1, 1)

        def pair(i, carry):
            for b in range(2):
                j = i * 2 + b
                pltpu.make_async_copy(
                    tt_hbm.at[:, pl.ds(bid_of(j) * BV, BV)], ibuf[b], lsem[b]).wait()

                @pl.when(i >= 1)
                def _wait_prev_store():
                    store(j - 2, b).wait()

                transpose(b)

                @pl.when(i < n_blocks // 2 - 1)
                def _next_load():
                    load_start(j + 2, b)

                store(j, b).start()
            return carry

        lax.fori_loop(0, n_blocks // 2, pair, 0)

        for b in range(2):
            store(n_blocks - 2 + b, b).wait()

        @pl.when(wid == 0)
        def _tail():
            pltpu.sync_copy(tail_hbm, tbuf)
            for u in range(tail_n):
                for c in range(D_MODEL // 16):
                    o0[u, pl.ds(c * 16, 16)] = (
                        tbuf[u, pl.ds(c * 16, 16)] * SCALE)
            pltpu.sync_copy(o0.at[pl.ds(0, tail_n), pl.ds(0, D_PAD)],
                            out_hbm.at[pl.ds(tail_v0, tail_n)])

    return fmt


def _make_sc_lookup(n_chunks_total: int):
    n_chunks = n_chunks_total // NW
    assert n_chunks * NW == n_chunks_total and n_chunks % 2 == 0
    n_pairs = n_chunks // 2

    mesh = plsc.VectorSubcoreMesh(core_axis_name="c", subcore_axis_name="s")

    @pl.kernel(
        out_type=jax.ShapeDtypeStruct((n_chunks_total * CHUNK, D_MODEL),
                                      jnp.float32),
        mesh=mesh,
        scratch_types=[
            pltpu.VMEM((n_chunks, CHUNK), jnp.int32),
            pltpu.VMEM((CHUNK, D_PAD), jnp.float32),
            pltpu.VMEM((CHUNK, D_PAD), jnp.float32),
            pltpu.VMEM((CHUNK, D_MODEL), jnp.float32),
            pltpu.VMEM((CHUNK, D_MODEL), jnp.float32),
            pltpu.SemaphoreType.DMA,
            pltpu.SemaphoreType.DMA,
            pltpu.SemaphoreType.DMA,
            pltpu.SemaphoreType.DMA,
        ],
    )
    def lookup(idx_hbm, table_hbm, out_hbm,
               idx_v, g0, g1, s0, s1, gsem0, gsem1, ssem0, ssem1):
        wid = lax.axis_index("s") * NUM_CORES + lax.axis_index("c")

        pltpu.sync_copy(idx_hbm.at[pl.ds(wid * n_chunks, n_chunks)], idx_v)

        gbuf = (g0, g1)
        sbuf = (s0, s1)
        gsem = (gsem0, gsem1)
        ssem = (ssem0, ssem1)

        def gather_start(j, b):
            pltpu.make_async_copy(
                table_hbm.at[idx_v.at[j]], gbuf[b], gsem[b]).start()

        def compact(b):
            src, dst = gbuf[b], sbuf[b]

            def body(r, carry):
                base = r * 8
                for k in range(8):
                    for c in range(D_MODEL // 16):
                        v = src[base + k, pl.ds(c * 16, 16)]
                        dst[base + k, pl.ds(c * 16, 16)] = v
                return carry

            lax.fori_loop(0, CHUNK // 8, body, 0)

        def scatter(j, b):
            return pltpu.make_async_copy(
                sbuf[b],
                out_hbm.at[pl.ds((wid * n_chunks + j) * CHUNK, CHUNK)],
                ssem[b])

        gather_start(0, 0)
        gather_start(1, 1)

        def pair(i, carry):
            for b in range(2):
                j = i * 2 + b
                pltpu.make_async_copy(
                    table_hbm.at[idx_v.at[j]], gbuf[b], gsem[b]).wait()

                @pl.when(i >= 1)
                def _wait_prev_scatter():
                    scatter(j - 2, b).wait()

                compact(b)

                @pl.when(i < n_pairs - 1)
                def _next_gather():
                    gather_start(j + 2, b)

                scatter(j, b).start()
            return carry

        lax.fori_loop(0, n_pairs, pair, 0)

        for b in range(2):
            scatter(n_chunks - 2 + b, b).wait()

    return lookup


def kernel(x, table):
    b, s = x.shape
    n = b * s
    vocab, d = table.shape
    assert d == D_MODEL and n % (NW * CHUNK * 2) == 0, (b, s, table.shape)
    idx = x.reshape(n // CHUNK, CHUNK).astype(jnp.int32)
    fmt = _make_sc_format(vocab)
    tail_v0 = (vocab // BV) * BV
    tablep = fmt(table.T, table[tail_v0:])
    lookup = _make_sc_lookup(n // CHUNK)
    out = lookup(idx, tablep)
    return out.reshape(b, s, D_MODEL)


# speedup vs baseline: 2.5088x; 2.4607x over previous
import math

import jax
import jax.numpy as jnp
from jax import lax
from jax.experimental import pallas as pl
from jax.experimental.pallas import tpu as pltpu
from jax.experimental.pallas import tpu_sc as plsc

D_MODEL = 64
D_PAD = 128
SCALE = math.sqrt(D_MODEL)
CHUNK = 128

_info = plsc.get_sparse_core_info()
NUM_CORES = _info.num_cores
NUM_SUBCORES = _info.num_subcores
NW = NUM_CORES * NUM_SUBCORES


def _make_tc_format(vocab: int):
    BVT = 8192
    grid = (vocab + BVT - 1) // BVT

    def body(tt_ref, out_ref):
        blk = tt_ref[...]
        t = jnp.transpose(blk, (1, 0)) * SCALE
        out_ref[...] = jnp.concatenate(
            [t, jnp.zeros((BVT, D_PAD - D_MODEL), jnp.float32)], axis=1)

    return pl.pallas_call(
        body,
        grid=(grid,),
        in_specs=[pl.BlockSpec((D_MODEL, BVT), lambda i: (0, i))],
        out_specs=pl.BlockSpec((BVT, D_PAD), lambda i: (i, 0)),
        out_shape=jax.ShapeDtypeStruct((vocab, D_PAD), jnp.float32),
    )


def _make_sc_lookup(n_chunks_total: int):
    n_chunks = n_chunks_total // NW
    assert n_chunks * NW == n_chunks_total and n_chunks % 2 == 0
    n_pairs = n_chunks // 2

    mesh = plsc.VectorSubcoreMesh(core_axis_name="c", subcore_axis_name="s")

    @pl.kernel(
        out_type=jax.ShapeDtypeStruct((n_chunks_total * CHUNK, D_MODEL),
                                      jnp.float32),
        mesh=mesh,
        scratch_types=[
            pltpu.VMEM((n_chunks, CHUNK), jnp.int32),
            pltpu.VMEM((CHUNK, D_PAD), jnp.float32),
            pltpu.VMEM((CHUNK, D_PAD), jnp.float32),
            pltpu.VMEM((CHUNK, D_MODEL), jnp.float32),
            pltpu.VMEM((CHUNK, D_MODEL), jnp.float32),
            pltpu.SemaphoreType.DMA,
            pltpu.SemaphoreType.DMA,
            pltpu.SemaphoreType.DMA,
            pltpu.SemaphoreType.DMA,
        ],
    )
    def lookup(idx_hbm, table_hbm, out_hbm,
               idx_v, g0, g1, s0, s1, gsem0, gsem1, ssem0, ssem1):
        wid = lax.axis_index("s") * NUM_CORES + lax.axis_index("c")

        pltpu.sync_copy(idx_hbm.at[pl.ds(wid * n_chunks, n_chunks)], idx_v)

        gbuf = (g0, g1)
        sbuf = (s0, s1)
        gsem = (gsem0, gsem1)
        ssem = (ssem0, ssem1)

        def gather_start(j, b):
            pltpu.make_async_copy(
                table_hbm.at[idx_v.at[j]], gbuf[b], gsem[b]).start()

        def scale(b):
            src, dst = gbuf[b], sbuf[b]

            def body(r, carry):
                base = r * 8
                for k in range(8):
                    for c in range(D_MODEL // 16):
                        v = src[base + k, pl.ds(c * 16, 16)]
                        dst[base + k, pl.ds(c * 16, 16)] = v * SCALE
                return carry

            lax.fori_loop(0, CHUNK // 8, body, 0)

        def scatter(j, b):
            return pltpu.make_async_copy(
                sbuf[b],
                out_hbm.at[pl.ds((wid * n_chunks + j) * CHUNK, CHUNK)],
                ssem[b])

        gather_start(0, 0)
        gather_start(1, 1)

        def pair(i, carry):
            for b in range(2):
                j = i * 2 + b
                pltpu.make_async_copy(
                    table_hbm.at[idx_v.at[j]], gbuf[b], gsem[b]).wait()

                @pl.when(i >= 1)
                def _wait_prev_scatter():
                    scatter(j - 2, b).wait()

                scale(b)

                @pl.when(i < n_pairs - 1)
                def _next_gather():
                    gather_start(j + 2, b)

                scatter(j, b).start()
            return carry

        lax.fori_loop(0, n_pairs, pair, 0)

        for b in range(2):
            scatter(n_chunks - 2 + b, b).wait()

    return lookup


def kernel(x, table):
    b, s = x.shape
    n = b * s
    assert n % (NW * CHUNK * 2) == 0, (b, s)
    idx = x.reshape(n // CHUNK, CHUNK).astype(jnp.int32)
    vocab = table.shape[0]
    tablep = _make_tc_format(vocab)(table.T)
    lookup = _make_sc_lookup(n // CHUNK)
    out = lookup(idx, tablep)
    return out.reshape(b, s, D_MODEL)
